# Initial kernel scaffold; baseline (speedup 1.0000x reference)
import jax
import jax.numpy as jnp
from jax.experimental import pallas as pl


def kernel(x, neigh_orders):
    raise NotImplementedError("write your pallas kernel here")


# speedup vs baseline: 9.7592x; 9.7592x over previous
import functools

import jax
import jax.numpy as jnp
from jax import lax
from jax.experimental import pallas as pl
from jax.experimental.pallas import tpu as pltpu
from jax.experimental.pallas import tpu_sc as plsc

N_NODES = 100000
FEAT = 128
NW = 32
NPW_MAIN = 3128
NPW_LAST = N_NODES - 31 * NPW_MAIN
CH = 8
ROWS = 7 * CH

_mesh = plsc.VectorSubcoreMesh(core_axis_name="c", subcore_axis_name="s")


@functools.partial(
    pl.kernel,
    mesh=_mesh,
    compiler_params=pltpu.CompilerParams(needs_layout_passes=False),
    out_type=[
        jax.ShapeDtypeStruct((N_NODES, FEAT), jnp.float32),
        jax.ShapeDtypeStruct((N_NODES, FEAT), jnp.int32),
    ],
    scratch_types=[
        pltpu.VMEM((ROWS,), jnp.int32),
        pltpu.VMEM((ROWS, FEAT), jnp.float32),
        pltpu.VMEM((CH, FEAT), jnp.float32),
        pltpu.VMEM((CH, FEAT), jnp.int32),
        pltpu.SemaphoreType.DMA,
    ],
)
def _sc_pool(x_hbm, no_hbm, vals_hbm, idxs_hbm, idx_v, rows_v, vout_v, iout_v, sem):
    wid = lax.axis_index("s") * 2 + lax.axis_index("c")
    node0 = wid * NPW_MAIN
    n_chunks = jnp.where(wid < NW - 1, NPW_MAIN // CH, NPW_LAST // CH)

    iota = lax.iota(jnp.int32, 16)
    iota7 = iota * 7

    def chunk_body(g, _):
        node_base = node0 + g * CH
        pltpu.sync_copy(no_hbm.at[pl.ds(node_base * 7, ROWS)], idx_v)
        pltpu.async_copy(x_hbm.at[idx_v], rows_v, sem).wait()

        def node_body(b, _):
            row_off = b * 7
            for v in range(FEAT // 16):
                p = iota7 + (112 * v)
                bval = plsc.load_gather(
                    rows_v, [(p >> 7) + row_off, p & 127])
                bidx = jnp.zeros((16,), jnp.int32)
                for k in range(1, 7):
                    pk = iota7 + (112 * v + k)
                    gv = plsc.load_gather(
                        rows_v, [(pk >> 7) + row_off, pk & 127])
                    m = gv > bval
                    bval = jnp.maximum(bval, gv)
                    bidx = jnp.where(m, jnp.full((16,), k, jnp.int32), bidx)
                vout_v[b, pl.ds(16 * v, 16)] = bval
                iout_v[b, pl.ds(16 * v, 16)] = bidx
            return 0

        lax.fori_loop(0, CH, node_body, 0)
        pltpu.sync_copy(vout_v, vals_hbm.at[pl.ds(node_base, CH)])
        pltpu.sync_copy(iout_v, idxs_hbm.at[pl.ds(node_base, CH)])
        return 0

    lax.fori_loop(0, n_chunks, chunk_body, 0)


def kernel(x, neigh_orders):
    no32 = neigh_orders.astype(jnp.int32)
    vals, idxs = _sc_pool(x, no32)
    return (vals, idxs)
